# baseline (speedup 1.0000x reference)
import jax
import jax.numpy as jnp
from jax.experimental import pallas as pl

B, L, A = 128, 512, 1024


def _matvec_bias_kernel(d_ref, w_ref, b_ref, o_ref):
    d = d_ref[...]
    w = w_ref[:, 0, :]
    acc = jax.lax.dot_general(
        d, w,
        dimension_numbers=(((2,), (1,)), ((0,), (0,))),
        preferred_element_type=jnp.float32,
    )
    o_ref[...] = acc[:, None, :] + b_ref[...]


def kernel(dict, coef):
    w = coef[:, None, :A]
    bias = coef[:, None, A:]
    BB = 2
    grid = (B // BB,)
    out = pl.pallas_call(
        _matvec_bias_kernel,
        grid=grid,
        in_specs=[
            pl.BlockSpec((BB, L, A), lambda i: (i, 0, 0)),
            pl.BlockSpec((BB, 1, A), lambda i: (i, 0, 0)),
            pl.BlockSpec((BB, 1, 1), lambda i: (i, 0, 0)),
        ],
        out_specs=pl.BlockSpec((BB, 1, L), lambda i: (i, 0, 0)),
        out_shape=jax.ShapeDtypeStruct((B, 1, L), jnp.float32),
    )(dict, w, bias)
    return out.reshape(B, L, 1)


# speedup vs baseline: 1.0626x; 1.0626x over previous
import jax
import jax.numpy as jnp
from jax.experimental import pallas as pl

B, L, A = 128, 512, 1024


def _matvec_bias_kernel(d_ref, w_ref, b_ref, o_ref):
    d = d_ref[...]
    w = w_ref[:, :, :]
    acc = jnp.sum(d * w, axis=-1)
    o_ref[...] = acc[:, None, :] + b_ref[...]


def kernel(dict, coef):
    w = coef[:, None, :A]
    bias = coef[:, None, A:]
    BB = 2
    grid = (B // BB,)
    out = pl.pallas_call(
        _matvec_bias_kernel,
        grid=grid,
        in_specs=[
            pl.BlockSpec((BB, L, A), lambda i: (i, 0, 0)),
            pl.BlockSpec((BB, 1, A), lambda i: (i, 0, 0)),
            pl.BlockSpec((BB, 1, 1), lambda i: (i, 0, 0)),
        ],
        out_specs=pl.BlockSpec((BB, 1, L), lambda i: (i, 0, 0)),
        out_shape=jax.ShapeDtypeStruct((B, 1, L), jnp.float32),
    )(dict, w, bias)
    return out.reshape(B, L, 1)


# speedup vs baseline: 1.3207x; 1.2429x over previous
import jax
import jax.numpy as jnp
from jax.experimental import pallas as pl

B, L, A = 128, 512, 1024


def _matvec_bias_kernel(d_ref, w_ref, b_ref, o_ref):
    d = d_ref[...]
    w = w_ref[:, :, :]
    acc = jnp.sum(d * w, axis=-1)
    o_ref[...] = acc[:, None, :] + b_ref[...]


def kernel(dict, coef):
    w = coef[:, None, :A]
    bias = coef[:, None, A:]
    BB = 8
    grid = (B // BB,)
    out = pl.pallas_call(
        _matvec_bias_kernel,
        grid=grid,
        in_specs=[
            pl.BlockSpec((BB, L, A), lambda i: (i, 0, 0)),
            pl.BlockSpec((BB, 1, A), lambda i: (i, 0, 0)),
            pl.BlockSpec((BB, 1, 1), lambda i: (i, 0, 0)),
        ],
        out_specs=pl.BlockSpec((BB, 1, L), lambda i: (i, 0, 0)),
        out_shape=jax.ShapeDtypeStruct((B, 1, L), jnp.float32),
    )(dict, w, bias)
    return out.reshape(B, L, 1)
